# baseline (speedup 1.0000x reference)
import functools

import jax
import jax.numpy as jnp
from jax import lax
from jax.experimental import pallas as pl
from jax.experimental.pallas import tpu as pltpu
from jax.experimental.pallas import tpu_sc as plsc

DEFVAL = -1.0
NC, NS, L = 2, 16, 16
NW = NC * NS
LUT_SIZE = 128
KPAD = 112


@jax.jit
def _lookup(inp, keys, values):
    m, n = inp.shape
    nk = keys.shape[0]
    nb = n // NW
    rc = 8
    nchunk = m // rc
    mesh = plsc.VectorSubcoreMesh(core_axis_name="c", subcore_axis_name="s")

    @functools.partial(
        pl.kernel,
        out_type=jax.ShapeDtypeStruct((m, n), jnp.float32),
        mesh=mesh,
        compiler_params=pltpu.CompilerParams(
            needs_layout_passes=False,
            skip_device_barrier=True,
            disable_bounds_checks=True,
        ),
        scratch_types=[
            pltpu.VMEM((KPAD,), jnp.int32),
            pltpu.VMEM((KPAD,), jnp.float32),
            pltpu.VMEM((LUT_SIZE,), jnp.float32),
            pltpu.VMEM((rc, nb), jnp.int32),
            pltpu.VMEM((rc, nb), jnp.int32),
            pltpu.VMEM((rc, nb), jnp.float32),
            pltpu.VMEM((rc, nb), jnp.float32),
            pltpu.SemaphoreType.DMA,
            pltpu.SemaphoreType.DMA,
            pltpu.SemaphoreType.DMA,
            pltpu.SemaphoreType.DMA,
        ],
    )
    def body(in_hbm, keys_hbm, vals_hbm, out_hbm, kv, vv, lut,
             a0, a1, o0, o1, si0, si1, so0, so1):
        wid = lax.axis_index("s") * NC + lax.axis_index("c")
        col = wid * nb

        bufs, outs = (a0, a1), (o0, o1)
        si, so = (si0, si1), (so0, so1)

        def start_in(g, p):
            pltpu.async_copy(
                in_hbm.at[pl.ds(g * rc, rc), pl.ds(col, nb)], bufs[p], si[p]
            )

        def wait_in(p):
            pltpu.make_async_copy(
                in_hbm.at[pl.ds(0, rc), pl.ds(col, nb)], bufs[p], si[p]
            ).wait()

        def start_out(g, p):
            pltpu.async_copy(
                outs[p], out_hbm.at[pl.ds(g * rc, rc), pl.ds(col, nb)], so[p]
            )

        def wait_out(p):
            pltpu.make_async_copy(
                out_hbm.at[pl.ds(0, rc), pl.ds(col, nb)], outs[p], so[p]
            ).wait()

        start_in(0, 0)
        start_in(1, 1)

        pltpu.sync_copy(keys_hbm, kv.at[pl.ds(0, nk)])
        pltpu.sync_copy(vals_hbm, vv.at[pl.ds(0, nk)])
        for j in range(LUT_SIZE // L):
            lut[pl.ds(j * L, L)] = jnp.full((L,), DEFVAL, jnp.float32)
        lane = lax.iota(jnp.int32, L)
        for j in range(KPAD // L):
            k_vec = kv[pl.ds(j * L, L)]
            v_vec = vv[pl.ds(j * L, L)]
            if (j + 1) * L <= nk:
                plsc.store_scatter(lut, [k_vec], v_vec)
            else:
                plsc.store_scatter(lut, [k_vec], v_vec, mask=lane < (nk - j * L))

        def gather_chunk(p):
            @plsc.parallel_loop(0, rc, step=1, unroll=1)
            def _(r):
                for c in range(0, nb, L):
                    idx = bufs[p][r, pl.ds(c, L)]
                    outs[p][r, pl.ds(c, L)] = plsc.load_gather(lut, [idx])

        def chunk_step(g, p):
            wait_in(p)

            @pl.when(g >= 2)
            def _():
                wait_out(p)

            gather_chunk(p)
            start_out(g, p)

            @pl.when(g + 2 < nchunk)
            def _():
                start_in(g + 2, p)

        def chunk_body(g, _):
            @pl.when(lax.rem(g, 2) == 0)
            def _():
                chunk_step(g, 0)

            @pl.when(lax.rem(g, 2) == 1)
            def _():
                chunk_step(g, 1)

            return 0

        lax.fori_loop(0, nchunk, chunk_body, 0)
        wait_out(0)
        wait_out(1)

    return body(inp, keys, values)


def kernel(inputs, keys, values):
    return _lookup(inputs.T, keys, values.astype(jnp.float32)).T


# speedup vs baseline: 1.1210x; 1.1210x over previous
import functools

import jax
import jax.numpy as jnp
from jax import lax
from jax.experimental import pallas as pl
from jax.experimental.pallas import tpu as pltpu
from jax.experimental.pallas import tpu_sc as plsc

DEFVAL = -1.0
NC, NS, L = 2, 16, 16
NW = NC * NS
LUT_SIZE = 128
KPAD = 112


@jax.jit
def _lookup(inp, keys, values):
    m, n = inp.shape
    nk = keys.shape[0]
    nb = n // NW
    rc = 40
    nchunk = m // rc
    mesh = plsc.VectorSubcoreMesh(core_axis_name="c", subcore_axis_name="s")

    @functools.partial(
        pl.kernel,
        out_type=jax.ShapeDtypeStruct((m, n), jnp.float32),
        mesh=mesh,
        compiler_params=pltpu.CompilerParams(
            needs_layout_passes=False,
            skip_device_barrier=True,
            disable_bounds_checks=True,
        ),
        scratch_types=[
            pltpu.VMEM((KPAD,), jnp.int32),
            pltpu.VMEM((KPAD,), jnp.float32),
            pltpu.VMEM((LUT_SIZE,), jnp.float32),
            pltpu.VMEM((rc, nb), jnp.int32),
            pltpu.VMEM((rc, nb), jnp.int32),
            pltpu.VMEM((rc, nb), jnp.float32),
            pltpu.VMEM((rc, nb), jnp.float32),
            pltpu.SemaphoreType.DMA,
            pltpu.SemaphoreType.DMA,
            pltpu.SemaphoreType.DMA,
            pltpu.SemaphoreType.DMA,
        ],
    )
    def body(in_hbm, keys_hbm, vals_hbm, out_hbm, kv, vv, lut,
             a0, a1, o0, o1, si0, si1, so0, so1):
        wid = lax.axis_index("s") * NC + lax.axis_index("c")
        col = wid * nb

        bufs, outs = (a0, a1), (o0, o1)
        si, so = (si0, si1), (so0, so1)
        in_d, out_d = [None] * nchunk, [None] * nchunk

        def start_in(g):
            p = g % 2
            in_d[g] = pltpu.async_copy(
                in_hbm.at[pl.ds(g * rc, rc), pl.ds(col, nb)], bufs[p], si[p]
            )

        start_in(0)
        if nchunk > 1:
            start_in(1)

        pltpu.sync_copy(keys_hbm, kv.at[pl.ds(0, nk)])
        pltpu.sync_copy(vals_hbm, vv.at[pl.ds(0, nk)])
        for j in range(LUT_SIZE // L):
            lut[pl.ds(j * L, L)] = jnp.full((L,), DEFVAL, jnp.float32)
        lane = lax.iota(jnp.int32, L)
        for j in range(KPAD // L):
            k_vec = kv[pl.ds(j * L, L)]
            v_vec = vv[pl.ds(j * L, L)]
            if (j + 1) * L <= nk:
                plsc.store_scatter(lut, [k_vec], v_vec)
            else:
                plsc.store_scatter(lut, [k_vec], v_vec, mask=lane < (nk - j * L))

        def gather_chunk(p):
            @plsc.parallel_loop(0, rc, step=1, unroll=1)
            def _(r):
                for c in range(0, nb, L):
                    idx = bufs[p][r, pl.ds(c, L)]
                    outs[p][r, pl.ds(c, L)] = plsc.load_gather(lut, [idx])

        for g in range(nchunk):
            in_d[g].wait()
            if g >= 2:
                out_d[g - 2].wait()
            gather_chunk(g % 2)
            out_d[g] = pltpu.async_copy(
                outs[g % 2], out_hbm.at[pl.ds(g * rc, rc), pl.ds(col, nb)],
                so[g % 2],
            )
            if g + 2 < nchunk:
                start_in(g + 2)
        for g in range(max(0, nchunk - 2), nchunk):
            out_d[g].wait()

    return body(inp, keys, values)


def kernel(inputs, keys, values):
    return _lookup(inputs.T, keys, values.astype(jnp.float32)).T
